# Initial kernel scaffold; baseline (speedup 1.0000x reference)
import jax
import jax.numpy as jnp
from jax.experimental import pallas as pl


def kernel(input_tensor):
    raise NotImplementedError("write your pallas kernel here")


# speedup vs baseline: 533.7145x; 533.7145x over previous
import math

import jax
import jax.numpy as jnp
from jax.experimental import pallas as pl
from jax.experimental.pallas import tpu as pltpu

B, T, C, H, W = 2, 8, 96, 224, 224
NF = T - 1
S = H * W
ROWS = 8
SCHUNK = S // ROWS
NREG = 64
NREL = 169
NBINS = 100
RS = 8
NEG_INF = float("-inf")


def _k1_body(x_ref, fmin_ref, fmax_ref, pmin_ref, pmax_ref, prev_ref):
    t = pl.program_id(1)
    s = pl.program_id(2)
    cur = x_ref[0, 0]
    r = cur.reshape(C // 8, 8, SCHUNK // 128, 128)
    cmin = jnp.min(r, axis=(0, 2))
    cmax = jnp.max(r, axis=(0, 2))

    @pl.when(s == 0)
    def _():
        fmin_ref[0, 0] = cmin
        fmax_ref[0, 0] = cmax

    @pl.when(s != 0)
    def _():
        fmin_ref[0, 0] = jnp.minimum(fmin_ref[0, 0], cmin)
        fmax_ref[0, 0] = jnp.maximum(fmax_ref[0, 0], cmax)

    @pl.when(jnp.logical_and(t == 0, s == 0))
    def _():
        pmin_ref[0, 0] = jnp.full((8, 128), jnp.inf, jnp.float32)
        pmax_ref[0, 0] = jnp.full((8, 128), -jnp.inf, jnp.float32)

    @pl.when(t != 0)
    def _():
        prod = prev_ref[:, pl.ds(s * SCHUNK, SCHUNK)] * cur
        pr = prod.reshape(C // 8, 8, SCHUNK // 128, 128)
        pmin_c = jnp.min(pr, axis=(0, 2))
        pmax_c = jnp.max(pr, axis=(0, 2))
        @pl.when(jnp.logical_and(t == 1, s == 0))
        def _():
            pmin_ref[0, 0] = pmin_c
            pmax_ref[0, 0] = pmax_c
        @pl.when(jnp.logical_or(t != 1, s != 0))
        def _():
            pmin_ref[0, 0] = jnp.minimum(pmin_ref[0, 0], pmin_c)
            pmax_ref[0, 0] = jnp.maximum(pmax_ref[0, 0], pmax_c)

    prev_ref[:, pl.ds(s * SCHUNK, SCHUNK)] = cur


def _k1(x4):
    nchunk = S // SCHUNK
    grid = (B, T, nchunk)
    spec_x = pl.BlockSpec((1, 1, C, SCHUNK), lambda b, t, s: (b, t, 0, s))
    spec_f = pl.BlockSpec((1, 1, 8, 128), lambda b, t, s: (b, t, 0, 0))
    spec_p = pl.BlockSpec((1, 1, 8, 128), lambda b, t, s: (b, 0, 0, 0))
    out_shape = [
        jax.ShapeDtypeStruct((B, T, 8, 128), jnp.float32),
        jax.ShapeDtypeStruct((B, T, 8, 128), jnp.float32),
        jax.ShapeDtypeStruct((B, 1, 8, 128), jnp.float32),
        jax.ShapeDtypeStruct((B, 1, 8, 128), jnp.float32),
    ]
    return pl.pallas_call(
        _k1_body,
        grid=grid,
        in_specs=[spec_x],
        out_specs=[spec_f, spec_f, spec_p, spec_p],
        out_shape=out_shape,
        scratch_shapes=[pltpu.VMEM((C, S), jnp.float32)],
        compiler_params=pltpu.CompilerParams(
            dimension_semantics=("arbitrary", "arbitrary", "arbitrary")),
    )(x4)


def _k2_body(x_ref, edges_ref, acc_ref, prev_ref):
    b = pl.program_id(0)
    t = pl.program_id(1)
    s = pl.program_id(2)
    cur = x_ref[0, 0]

    @pl.when(jnp.logical_and(t == 0, s == 0))
    def _():
        acc_ref[0] = jnp.zeros((3, NBINS, 8, 128), jnp.float32)

    def count_ge(vals, hist_idx):
        def body(i, _):
            e = edges_ref[b, hist_idx, i]
            mask = (vals >= e).astype(jnp.float32)
            ssum = jnp.sum(mask.reshape(C // 8, 8, SCHUNK // 128, 128),
                           axis=(0, 2))
            acc_ref[0, hist_idx, i] = acc_ref[0, hist_idx, i] + ssum
            return 0
        jax.lax.fori_loop(0, NBINS, body, 0)

    @pl.when(t <= T - 2)
    def _():
        count_ge(cur, 0)

    @pl.when(t >= 1)
    def _():
        count_ge(cur, 1)
        prod = prev_ref[:, pl.ds(s * SCHUNK, SCHUNK)] * cur
        count_ge(prod, 2)

    prev_ref[:, pl.ds(s * SCHUNK, SCHUNK)] = cur


def _k2(x4, edges):
    nchunk = S // SCHUNK
    grid = (B, T, nchunk)
    spec_x = pl.BlockSpec((1, 1, C, SCHUNK), lambda b, t, s: (b, t, 0, s))
    spec_e = pl.BlockSpec(memory_space=pltpu.SMEM)
    spec_o = pl.BlockSpec((1, 3, NBINS, 8, 128), lambda b, t, s: (b, 0, 0, 0, 0))
    return pl.pallas_call(
        _k2_body,
        grid=grid,
        in_specs=[spec_x, spec_e],
        out_specs=spec_o,
        out_shape=jax.ShapeDtypeStruct((B, 3, NBINS, 8, 128), jnp.float32),
        scratch_shapes=[pltpu.VMEM((C, S), jnp.float32)],
        compiler_params=pltpu.CompilerParams(
            dimension_semantics=("arbitrary", "arbitrary", "arbitrary")),
    )(x4, edges)


def _k3_body(x0_ref, x1_ref, add_ref, mm_ref):
    b = pl.program_id(0)
    add = add_ref[b]
    terms = (x1_ref[0, 0] - x0_ref[0, 0]) + add
    mm_col = jnp.sum(terms, axis=1)
    mm_ref[0, 0, 0] = mm_col[None, :]


def _k3(xt_view, add_vec):
    nchunk = S // SCHUNK
    grid = (B, NF, nchunk)
    spec_x0 = pl.BlockSpec((1, 1, SCHUNK, C), lambda b, f, s: (b, f, s, 0))
    spec_x1 = pl.BlockSpec((1, 1, SCHUNK, C), lambda b, f, s: (b, f + 1, s, 0))
    spec_a = pl.BlockSpec(memory_space=pltpu.SMEM)
    spec_o = pl.BlockSpec((1, 1, 1, 1, SCHUNK), lambda b, f, s: (b, f, s, 0, 0))
    return pl.pallas_call(
        _k3_body,
        grid=grid,
        in_specs=[spec_x0, spec_x1, spec_a],
        out_specs=spec_o,
        out_shape=jax.ShapeDtypeStruct((B, NF, ROWS, 1, SCHUNK), jnp.float32),
        compiler_params=pltpu.CompilerParams(
            dimension_semantics=("arbitrary", "arbitrary", "arbitrary")),
    )(xt_view, xt_view, add_vec)


def _k4_body(mm_ref, pmap_ref, out_ref):
    vals = mm_ref[0]
    pmap = pmap_ref[...]
    iota_row = jax.lax.broadcasted_iota(jnp.int32, (1, NREG), 1)
    iota_col = jax.lax.broadcasted_iota(jnp.int32, (NREG, 1), 0)
    BIG = jnp.int32(2 ** 30)

    def sel_body(i, carry):
        live, cyr, cxr, cyc, cxc = carry
        m = jnp.max(live)
        eq = live == m
        p_sel = jnp.min(jnp.where(eq, pmap, BIG))
        live = jnp.where(pmap == p_sel, NEG_INF, live)
        yy = p_sel // NREG
        xx = p_sel % NREG
        cy = (yy * RS + RS // 2).astype(jnp.float32)
        cx = (xx * RS + RS // 2).astype(jnp.float32)
        cyr = cyr + jnp.where(iota_row == i, cy, 0.0)
        cxr = cxr + jnp.where(iota_row == i, cx, 0.0)
        cyc = cyc + jnp.where(iota_col == i, cy, 0.0)
        cxc = cxc + jnp.where(iota_col == i, cx, 0.0)
        return live, cyr, cxr, cyc, cxc

    z_r = jnp.zeros((1, NREG), jnp.float32)
    z_c = jnp.zeros((NREG, 1), jnp.float32)
    _, cyr, cxr, cyc, cxc = jax.lax.fori_loop(
        0, NREG, sel_body, (vals, z_r, z_r, z_c, z_c))

    mean_y = jnp.sum(cyr) / NREG
    mean_x = jnp.sum(cxr) / NREG
    dy = cyr - mean_y
    dx = cxr - mean_x
    sxx = jnp.sum(dy * dy) / (NREG - 1)
    sxy = jnp.sum(dy * dx) / (NREG - 1)
    syy = jnp.sum(dx * dx) / (NREG - 1)
    c00 = sxx + 1e-6
    c11 = syy + 1e-6
    det = c00 * c11 - sxy * sxy
    i00 = c11 / det
    i01 = -sxy / det
    i11 = c00 / det
    ddy = cyr - cyc
    ddx = cxr - cxc
    m0 = ddy * i00 + ddx * i01
    m1 = ddy * i01 + ddx * i11
    maha = jnp.sqrt(m0 * ddy) + jnp.sqrt(m1 * ddx)
    key = jnp.where(jnp.isnan(maha), float("inf"), maha)

    idx64 = (jax.lax.broadcasted_iota(jnp.int32, (NREG, NREG), 0) * NREG
             + jax.lax.broadcasted_iota(jnp.int32, (NREG, NREG), 1))

    def rel_body(i, carry):
        live, acc = carry
        m = jnp.min(live)
        eq = live == m
        sel = jnp.min(jnp.where(eq, idx64, BIG))
        hit = idx64 == sel
        acc = acc + jnp.where(hit, 1.0, 0.0)
        live = jnp.where(hit, float("inf"), live)
        return live, acc

    acc0 = jnp.zeros((NREG, NREG), jnp.float32)
    _, acc = jax.lax.fori_loop(0, NREL, rel_body, (key, acc0))
    out_ref[0] = acc


def _k4(mm_rows, pmap):
    grid = (B * NF,)
    spec_mm = pl.BlockSpec((1, ROWS, SCHUNK), lambda f: (f, 0, 0))
    spec_pm = pl.BlockSpec((ROWS, SCHUNK), lambda f: (0, 0))
    spec_o = pl.BlockSpec((1, NREG, NREG), lambda f: (f, 0, 0))
    return pl.pallas_call(
        _k4_body,
        grid=grid,
        in_specs=[spec_mm, spec_pm],
        out_specs=spec_o,
        out_shape=jax.ShapeDtypeStruct((B * NF, NREG, NREG), jnp.float32),
        compiler_params=pltpu.CompilerParams(
            dimension_semantics=("arbitrary",)),
    )(mm_rows, pmap)


def _edges_from_range(vmin, vmax):
    lo = jnp.where(vmax - vmin == 0, vmin - 0.5, vmin)
    hi = jnp.where(vmax - vmin == 0, vmax + 0.5, vmax)
    step = jax.lax.iota(jnp.float32, NBINS) / jnp.float32(NBINS)
    return lo * (1 - step) + hi * step


def kernel(input_tensor):
    x4 = input_tensor.reshape(B, T, C, S)

    fmin, fmax, pmin, pmax = _k1(x4)
    xt_min = jnp.min(fmin[:, :T - 1], axis=(1, 2, 3))
    xt_max = jnp.max(fmax[:, :T - 1], axis=(1, 2, 3))
    xt1_min = jnp.min(fmin[:, 1:], axis=(1, 2, 3))
    xt1_max = jnp.max(fmax[:, 1:], axis=(1, 2, 3))
    pr_min = jnp.min(pmin[:, 0], axis=(1, 2))
    pr_max = jnp.max(pmax[:, 0], axis=(1, 2))

    edges = jnp.stack([
        jnp.stack([_edges_from_range(xt_min[b], xt_max[b]) for b in range(B)]),
        jnp.stack([_edges_from_range(xt1_min[b], xt1_max[b]) for b in range(B)]),
        jnp.stack([_edges_from_range(pr_min[b], pr_max[b]) for b in range(B)]),
    ], axis=1)

    acc = _k2(x4, edges)
    lane_counts = jnp.concatenate(
        [acc[:, :, :-1] - acc[:, :, 1:], acc[:, :, -1:]], axis=2)
    counts = jnp.sum(lane_counts, axis=(3, 4))

    nelem = (T - 1) * C * S
    adds = []
    for b in range(B):
        joint_hist = counts[b, 2]
        p_x = counts[b, 0] / nelem
        p_x1 = counts[b, 1] / nelem
        p_joint = joint_hist / joint_hist.sum()
        p_x = p_x / p_x.sum()
        p_x1 = p_x1 / p_x1.sum()
        mutual_info = (p_joint * (jnp.log(p_joint + 1e-10)
                                  - jnp.log(p_x * p_x1 + 1e-10))).sum()
        adds.append(math.e ** 0.8 / jnp.tanh(jnp.log(mutual_info)))
    add_vec = jnp.stack(adds).astype(jnp.float32)

    xt_view = jnp.swapaxes(x4, 2, 3)
    mm = _k3(xt_view, add_vec)
    mm_rows = mm.reshape(B * NF, ROWS, SCHUNK)

    q = jnp.arange(S, dtype=jnp.int32)
    h = q // W
    w = q % W
    pmap = (((h // RS) * (W // RS) + w // RS) * (RS * RS)
            + (h % RS) * RS + (w % RS)).reshape(ROWS, SCHUNK)

    out = _k4(mm_rows, pmap)
    return out.reshape(B, NF, NREG, NREG)
